# Initial kernel scaffold; baseline (speedup 1.0000x reference)
import functools

import jax
import jax.numpy as jnp
from jax import lax
from jax.experimental import pallas as pl
from jax.experimental.pallas import tpu as pltpu
from jax.experimental.pallas import tpu_sc as plsc

N_TOKEN = 1000000
D_EMBED = 128
D_PROJ = 128
CUT1 = 20000
CUT2 = 200000
SCALE = float(D_PROJ) ** 0.5

N = 16384 * 20
NC, NS = 2, 16
NW = NC * NS
B_PER_W = N // NW
CHUNK = 128
N_CHUNKS = B_PER_W // CHUNK

_BLK_A = 400
_BLK_B = 100
_N_A = CUT1 // _BLK_A
_N_B = (CUT2 - CUT1) // (4 * _BLK_B)


def _preproject_body(emb0_ref, emb1p_ref, w0_ref, w1_ref, out_ref):
    i = pl.program_id(0)

    @pl.when(i < _N_A)
    def _():
        out_ref[...] = jnp.dot(emb0_ref[...], w0_ref[...],
                               preferred_element_type=jnp.float32)

    @pl.when(i >= _N_A)
    def _():
        r = jnp.dot(emb1p_ref[...], w1_ref[...],
                    preferred_element_type=jnp.float32)
        out_ref[...] = r.reshape(_BLK_A, D_PROJ)


def _preproject(emb_0, emb_1, proj_0, proj_1):
    emb1p = emb_1.reshape((CUT2 - CUT1) // 4, 4 * 32)
    w0 = proj_0.T * SCALE
    p1t = proj_1.T * SCALE
    w1 = jnp.zeros((128, 4 * D_PROJ), jnp.float32)
    for j in range(4):
        w1 = w1.at[32 * j:32 * (j + 1), 128 * j:128 * (j + 1)].set(p1t)

    return pl.pallas_call(
        _preproject_body,
        grid=(_N_A + _N_B,),
        in_specs=[
            pl.BlockSpec((_BLK_A, 128), lambda i: (jnp.minimum(i, _N_A - 1), 0)),
            pl.BlockSpec((_BLK_B, 128), lambda i: (jnp.clip(i - _N_A, 0, _N_B - 1), 0)),
            pl.BlockSpec((128, 128), lambda i: (0, 0)),
            pl.BlockSpec((128, 4 * D_PROJ), lambda i: (0, 0)),
        ],
        out_specs=pl.BlockSpec((_BLK_A, D_PROJ), lambda i: (i, 0)),
        out_shape=jax.ShapeDtypeStruct((CUT2, D_PROJ), jnp.float32),
    )(emb_0, emb1p, w0, w1)


def _sc_gather_body(idx01_hbm, idx2_hbm, pt_hbm, emb2_hbm,
                    rows01_hbm, rows2_hbm,
                    idx01_v, idx2_v, rows01_v, rows2_v, sem1, sem2):
    wid = lax.axis_index("s") * NC + lax.axis_index("c")
    base = wid * B_PER_W

    def body(c, carry):
        off = base + c * CHUNK
        pltpu.sync_copy(idx01_hbm.at[pl.ds(off, CHUNK)], idx01_v)
        pltpu.sync_copy(idx2_hbm.at[pl.ds(off, CHUNK)], idx2_v)
        cp1 = pltpu.async_copy(pt_hbm.at[idx01_v], rows01_v, sem1)
        cp2 = pltpu.async_copy(emb2_hbm.at[idx2_v], rows2_v, sem2)
        cp1.wait()
        cp2.wait()
        pltpu.sync_copy(rows01_v, rows01_hbm.at[pl.ds(off, CHUNK)])
        pltpu.sync_copy(rows2_v, rows2_hbm.at[pl.ds(off, CHUNK)])
        return carry

    lax.fori_loop(0, N_CHUNKS, body, 0)


def _sc_gather(idx01, idx2, pt, emb_2):
    mesh = plsc.VectorSubcoreMesh(core_axis_name="c", subcore_axis_name="s")
    f = functools.partial(
        pl.kernel,
        mesh=mesh,
        out_type=[
            jax.ShapeDtypeStruct((N, D_PROJ), jnp.float32),
            jax.ShapeDtypeStruct((N, 8), jnp.float32),
        ],
        scratch_types=[
            pltpu.VMEM((CHUNK,), jnp.int32),
            pltpu.VMEM((CHUNK,), jnp.int32),
            pltpu.VMEM((CHUNK, D_PROJ), jnp.float32),
            pltpu.VMEM((CHUNK, 8), jnp.float32),
            pltpu.SemaphoreType.DMA,
            pltpu.SemaphoreType.DMA,
        ],
    )(_sc_gather_body)
    return f(idx01, idx2, pt, emb_2)


_CBLK = 1024
_N_CB = N // _CBLK


def _combine_body(tok_ref, rows01_ref, rows2_ref, w2_ref, out_ref):
    mask = tok_ref[0, 0, :] >= CUT2
    c2 = jnp.dot(rows2_ref[...], w2_ref[...],
                 preferred_element_type=jnp.float32)
    out_ref[...] = jnp.where(mask[:, None], c2, rows01_ref[...])


def _combine(tok3, rows01, rows2, w2):
    return pl.pallas_call(
        _combine_body,
        grid=(_N_CB,),
        in_specs=[
            pl.BlockSpec((1, 1, _CBLK), lambda i: (i, 0, 0)),
            pl.BlockSpec((_CBLK, D_PROJ), lambda i: (i, 0)),
            pl.BlockSpec((_CBLK, 8), lambda i: (i, 0)),
            pl.BlockSpec((8, D_PROJ), lambda i: (0, 0)),
        ],
        out_specs=pl.BlockSpec((_CBLK, D_PROJ), lambda i: (i, 0)),
        out_shape=jax.ShapeDtypeStruct((N, D_PROJ), jnp.float32),
    )(tok3, rows01, rows2, w2)


def kernel(inp, emb_0, emb_1, emb_2, proj_0, proj_1, proj_2):
    inp_flat = inp.reshape(-1).astype(jnp.int32)
    is2 = inp_flat >= CUT2
    idx01 = jnp.where(is2, 0, inp_flat)
    idx2 = jnp.where(is2, inp_flat - CUT2, 0)

    pt = _preproject(emb_0, emb_1, proj_0, proj_1)
    rows01, rows2 = _sc_gather(idx01, idx2, pt, emb_2)

    tok3 = inp_flat.reshape(_N_CB, 1, _CBLK)
    w2 = proj_2.T * SCALE
    out = _combine(tok3, rows01, rows2, w2)
    return out.reshape(inp.shape + (D_PROJ,))


# speedup vs baseline: 2.0715x; 2.0715x over previous
import functools

import jax
import jax.numpy as jnp
from jax import lax
from jax.experimental import pallas as pl
from jax.experimental.pallas import tpu as pltpu
from jax.experimental.pallas import tpu_sc as plsc

N_TOKEN = 1000000
D_EMBED = 128
D_PROJ = 128
CUT1 = 20000
CUT2 = 200000
SCALE = float(D_PROJ) ** 0.5

N = 16384 * 20
NC, NS = 2, 16
NW = NC * NS
B_PER_W = N // NW
CHUNK = 128
N_CHUNKS = B_PER_W // CHUNK

_BLK_A = 800
_BLK_B = 200
_N_A = CUT1 // _BLK_A
_N_B = (CUT2 - CUT1) // (4 * _BLK_B)


def _preproject_body(emb0_ref, emb1p_ref, w0_ref, w1_ref, out_ref):
    i = pl.program_id(0)

    @pl.when(i < _N_A)
    def _():
        out_ref[...] = jnp.dot(emb0_ref[...], w0_ref[...],
                               preferred_element_type=jnp.float32)

    @pl.when(i >= _N_A)
    def _():
        r = jnp.dot(emb1p_ref[...], w1_ref[...],
                    preferred_element_type=jnp.float32)
        out_ref[...] = r.reshape(_BLK_A, D_PROJ)


def _preproject(emb_0, emb_1, proj_0, proj_1):
    emb1p = emb_1.reshape((CUT2 - CUT1) // 4, 4 * 32)
    w0 = proj_0.T * SCALE
    p1t = proj_1.T * SCALE
    w1 = jnp.zeros((128, 4 * D_PROJ), jnp.float32)
    for j in range(4):
        w1 = w1.at[32 * j:32 * (j + 1), 128 * j:128 * (j + 1)].set(p1t)

    return pl.pallas_call(
        _preproject_body,
        grid=(_N_A + _N_B,),
        in_specs=[
            pl.BlockSpec((_BLK_A, 128), lambda i: (jnp.minimum(i, _N_A - 1), 0)),
            pl.BlockSpec((_BLK_B, 128), lambda i: (jnp.clip(i - _N_A, 0, _N_B - 1), 0)),
            pl.BlockSpec((128, 128), lambda i: (0, 0)),
            pl.BlockSpec((128, 4 * D_PROJ), lambda i: (0, 0)),
        ],
        out_specs=pl.BlockSpec((_BLK_A, D_PROJ), lambda i: (i, 0)),
        out_shape=jax.ShapeDtypeStruct((CUT2, D_PROJ), jnp.float32),
    )(emb_0, emb1p, w0, w1)


def _sc_gather_body(idx01_hbm, idx2_hbm, pt_hbm, emb2_hbm,
                    rows01_hbm, rows2_hbm,
                    idx01_v, idx2_v, rows01_v, rows2_v, sem1, sem2):
    wid = lax.axis_index("s") * NC + lax.axis_index("c")
    base = wid * B_PER_W

    def body(c, carry):
        off = base + c * CHUNK
        pltpu.sync_copy(idx01_hbm.at[pl.ds(off, CHUNK)], idx01_v)
        pltpu.sync_copy(idx2_hbm.at[pl.ds(off, CHUNK)], idx2_v)
        cp1 = pltpu.async_copy(pt_hbm.at[idx01_v], rows01_v, sem1)
        cp2 = pltpu.async_copy(emb2_hbm.at[idx2_v], rows2_v, sem2)
        cp1.wait()
        cp2.wait()
        pltpu.sync_copy(rows01_v, rows01_hbm.at[pl.ds(off, CHUNK)])
        pltpu.sync_copy(rows2_v, rows2_hbm.at[pl.ds(off, CHUNK)])
        return carry

    lax.fori_loop(0, N_CHUNKS, body, 0)


def _sc_gather(idx01, idx2, pt, emb_2):
    mesh = plsc.VectorSubcoreMesh(core_axis_name="c", subcore_axis_name="s")
    f = functools.partial(
        pl.kernel,
        mesh=mesh,
        out_type=[
            jax.ShapeDtypeStruct((N, D_PROJ), jnp.float32),
            jax.ShapeDtypeStruct((N, 8), jnp.float32),
        ],
        scratch_types=[
            pltpu.VMEM((CHUNK,), jnp.int32),
            pltpu.VMEM((CHUNK,), jnp.int32),
            pltpu.VMEM((CHUNK, D_PROJ), jnp.float32),
            pltpu.VMEM((CHUNK, 8), jnp.float32),
            pltpu.SemaphoreType.DMA,
            pltpu.SemaphoreType.DMA,
        ],
        compiler_params=pltpu.CompilerParams(use_tc_tiling_on_sc=False),
    )(_sc_gather_body)
    return f(idx01, idx2, pt, emb_2)


_CBLK = 1024
_N_CB = N // _CBLK


def _combine_body(tok_ref, rows01_ref, rows2_ref, w2_ref, out_ref):
    mask = tok_ref[...] >= CUT2
    c2 = jnp.dot(rows2_ref[...], w2_ref[...],
                 preferred_element_type=jnp.float32)
    out_ref[...] = jnp.where(mask, c2, rows01_ref[...])


def _combine(tok2, rows01, rows2, w2):
    return pl.pallas_call(
        _combine_body,
        grid=(_N_CB,),
        in_specs=[
            pl.BlockSpec((_CBLK, 1), lambda i: (i, 0)),
            pl.BlockSpec((_CBLK, D_PROJ), lambda i: (i, 0)),
            pl.BlockSpec((_CBLK, 8), lambda i: (i, 0)),
            pl.BlockSpec((8, D_PROJ), lambda i: (0, 0)),
        ],
        out_specs=pl.BlockSpec((_CBLK, D_PROJ), lambda i: (i, 0)),
        out_shape=jax.ShapeDtypeStruct((N, D_PROJ), jnp.float32),
    )(tok2, rows01, rows2, w2)


def kernel(inp, emb_0, emb_1, emb_2, proj_0, proj_1, proj_2):
    inp_flat = inp.reshape(-1).astype(jnp.int32)
    is2 = inp_flat >= CUT2
    idx01 = jnp.where(is2, 0, inp_flat)
    idx2 = jnp.where(is2, inp_flat - CUT2, 0)

    pt = _preproject(emb_0, emb_1, proj_0, proj_1)
    rows01, rows2 = _sc_gather(idx01, idx2, pt, emb_2)

    tok2 = inp_flat.reshape(N, 1)
    w2 = proj_2.T * SCALE
    out = _combine(tok2, rows01, rows2, w2)
    return out.reshape(inp.shape + (D_PROJ,))
